# Initial kernel scaffold; baseline (speedup 1.0000x reference)
import jax
import jax.numpy as jnp
from jax import lax
from jax.experimental import pallas as pl
from jax.experimental.pallas import tpu as pltpu
from jax.experimental.pallas import tpu_sc as plsc

H = 64
NE = 32
NA = 16
NV = NE * NA

NC = 2
NS = 16
NW = NC * NS

B_ROWS = 16384 * 50
ROWS_PER_W = B_ROWS // NW
IDX_MINOR = 128
IDX_ROWS_PER_W = ROWS_PER_W // IDX_MINOR
CHUNK_IDX_ROWS = 4
CHUNK_ROWS = CHUNK_IDX_ROWS * IDX_MINOR
N_CHUNKS = ROWS_PER_W // CHUNK_ROWS


def _prep_body(et_ref, at_ref, w_ref, b_ref, ent_ref, attr_ref, c_ref, idx_ref):
    e2 = jnp.dot(et_ref[...], w_ref[0:H, :],
                 preferred_element_type=jnp.float32)
    a2 = jnp.dot(at_ref[...], w_ref[H:2 * H, :],
                 preferred_element_type=jnp.float32)
    r = lax.broadcasted_iota(jnp.int32, (NV, NE), 0) // NA
    c = lax.broadcasted_iota(jnp.int32, (NV, NE), 1)
    oh_e = (r == c).astype(jnp.float32)
    r2 = lax.broadcasted_iota(jnp.int32, (NV, NA), 0) % NA
    c2 = lax.broadcasted_iota(jnp.int32, (NV, NA), 1)
    oh_a = (r2 == c2).astype(jnp.float32)
    c_ref[...] = (jnp.dot(oh_e, e2, preferred_element_type=jnp.float32)
                  + jnp.dot(oh_a, a2, preferred_element_type=jnp.float32)
                  + b_ref[...])
    idx_ref[...] = ent_ref[...] * NA + attr_ref[...]


def _sc_body(c_hbm, idx_hbm, out_hbm, idx_v, rows_v, gsem0, gsem1):
    wid = lax.axis_index("s") * NC + lax.axis_index("c")
    idx_row0 = wid * IDX_ROWS_PER_W
    out_row0 = wid * ROWS_PER_W

    pltpu.sync_copy(idx_hbm.at[pl.ds(idx_row0, IDX_ROWS_PER_W)], idx_v)

    def gathers(c, slot, sem):
        cps = []
        for j in range(CHUNK_IDX_ROWS):
            cps.append(pltpu.async_copy(
                c_hbm.at[idx_v.at[c * CHUNK_IDX_ROWS + j]],
                rows_v.at[slot, pl.ds(j * IDX_MINOR, IDX_MINOR)],
                sem))
        return cps

    def writeout(c, slot):
        pltpu.sync_copy(rows_v.at[slot],
                        out_hbm.at[pl.ds(out_row0 + c * CHUNK_ROWS,
                                         CHUNK_ROWS)])

    @pl.loop(0, N_CHUNKS // 2)
    def _(i):
        c0 = i * 2
        c1 = c0 + 1
        g0 = gathers(c0, 0, gsem0)
        g1 = gathers(c1, 1, gsem1)
        for cp in g0:
            cp.wait()
        writeout(c0, 0)
        for cp in g1:
            cp.wait()
        writeout(c1, 1)


def kernel(ent, attr, entity_table, attr_table, W, b):
    B, L = ent.shape
    ent32 = ent.astype(jnp.int32).reshape(B_ROWS // IDX_MINOR, IDX_MINOR)
    attr32 = attr.astype(jnp.int32).reshape(B_ROWS // IDX_MINOR, IDX_MINOR)
    b2 = b.reshape(1, H)

    c_tab, idx = pl.pallas_call(
        _prep_body,
        out_shape=(
            jax.ShapeDtypeStruct((NV, H), jnp.float32),
            jax.ShapeDtypeStruct((B_ROWS // IDX_MINOR, IDX_MINOR), jnp.int32),
        ),
    )(entity_table, attr_table, W, b2, ent32, attr32)

    mesh = plsc.VectorSubcoreMesh(core_axis_name="c", subcore_axis_name="s",
                                  num_cores=NC, num_subcores=NS)
    out_flat = pl.kernel(
        _sc_body,
        out_type=jax.ShapeDtypeStruct((B_ROWS, H), jnp.float32),
        mesh=mesh,
        scratch_types=[
            pltpu.VMEM((IDX_ROWS_PER_W, IDX_MINOR), jnp.int32),
            pltpu.VMEM((2, CHUNK_ROWS, H), jnp.float32),
            pltpu.SemaphoreType.DMA,
            pltpu.SemaphoreType.DMA,
        ],
    )(c_tab, idx)

    return out_flat.reshape(B, L, H)


# speedup vs baseline: 6.5975x; 6.5975x over previous
import jax
import jax.numpy as jnp
from jax import lax
from jax.experimental import pallas as pl
from jax.experimental.pallas import tpu as pltpu
from jax.experimental.pallas import tpu_sc as plsc

H = 64
NE = 32
NA = 16
NV = NE * NA

NC = 2
NS = 16
NW = NC * NS

B_ROWS = 16384 * 50
ROWS_PER_W = B_ROWS // NW
IDX_MINOR = 128
IDX_ROWS_PER_W = ROWS_PER_W // IDX_MINOR
CHUNK_IDX_ROWS = 4
CHUNK_ROWS = CHUNK_IDX_ROWS * IDX_MINOR
N_CHUNKS = ROWS_PER_W // CHUNK_ROWS


def _prep_body(et_ref, at_ref, w_ref, b_ref, ent_ref, attr_ref, c_ref, idx_ref):
    e2 = jnp.dot(et_ref[...], w_ref[0:H, :], precision=lax.Precision.HIGHEST,
                 preferred_element_type=jnp.float32)
    a2 = jnp.dot(at_ref[...], w_ref[H:2 * H, :], precision=lax.Precision.HIGHEST,
                 preferred_element_type=jnp.float32)
    r = lax.broadcasted_iota(jnp.int32, (NV, NE), 0) // NA
    c = lax.broadcasted_iota(jnp.int32, (NV, NE), 1)
    oh_e = (r == c).astype(jnp.float32)
    r2 = lax.broadcasted_iota(jnp.int32, (NV, NA), 0) % NA
    c2 = lax.broadcasted_iota(jnp.int32, (NV, NA), 1)
    oh_a = (r2 == c2).astype(jnp.float32)
    c_ref[...] = (jnp.dot(oh_e, e2, precision=lax.Precision.HIGHEST,
                          preferred_element_type=jnp.float32)
                  + jnp.dot(oh_a, a2, precision=lax.Precision.HIGHEST,
                            preferred_element_type=jnp.float32)
                  + b_ref[...])
    idx_ref[...] = ent_ref[...] * NA + attr_ref[...]


def _sc_body(c_hbm, idx_hbm, out_hbm, idx_v, rows_v, gsem0, gsem1):
    wid = lax.axis_index("s") * NC + lax.axis_index("c")
    idx_row0 = wid * IDX_ROWS_PER_W
    out_row0 = wid * ROWS_PER_W

    pltpu.sync_copy(idx_hbm.at[pl.ds(idx_row0, IDX_ROWS_PER_W)], idx_v)

    def gathers(c, slot, sem):
        cps = []
        for j in range(CHUNK_IDX_ROWS):
            cps.append(pltpu.async_copy(
                c_hbm.at[idx_v.at[c * CHUNK_IDX_ROWS + j]],
                rows_v.at[slot, pl.ds(j * IDX_MINOR, IDX_MINOR)],
                sem))
        return cps

    def writeout(c, slot):
        pltpu.sync_copy(rows_v.at[slot],
                        out_hbm.at[pl.ds(out_row0 + c * CHUNK_ROWS,
                                         CHUNK_ROWS)])

    @pl.loop(0, N_CHUNKS // 2)
    def _(i):
        c0 = i * 2
        c1 = c0 + 1
        g0 = gathers(c0, 0, gsem0)
        g1 = gathers(c1, 1, gsem1)
        for cp in g0:
            cp.wait()
        writeout(c0, 0)
        for cp in g1:
            cp.wait()
        writeout(c1, 1)


def kernel(ent, attr, entity_table, attr_table, W, b):
    B, L = ent.shape
    ent32 = ent.astype(jnp.int32).reshape(B_ROWS // IDX_MINOR, IDX_MINOR)
    attr32 = attr.astype(jnp.int32).reshape(B_ROWS // IDX_MINOR, IDX_MINOR)
    b2 = b.reshape(1, H)

    c_tab, idx = pl.pallas_call(
        _prep_body,
        out_shape=(
            jax.ShapeDtypeStruct((NV, H), jnp.float32),
            jax.ShapeDtypeStruct((B_ROWS // IDX_MINOR, IDX_MINOR), jnp.int32),
        ),
    )(entity_table, attr_table, W, b2, ent32, attr32)

    mesh = plsc.VectorSubcoreMesh(core_axis_name="c", subcore_axis_name="s",
                                  num_cores=NC, num_subcores=NS)
    out_flat = pl.kernel(
        _sc_body,
        out_type=jax.ShapeDtypeStruct((B_ROWS, H), jnp.float32),
        mesh=mesh,
        compiler_params=pltpu.CompilerParams(use_tc_tiling_on_sc=False),
        scratch_types=[
            pltpu.VMEM((IDX_ROWS_PER_W, IDX_MINOR), jnp.int32),
            pltpu.VMEM((2, CHUNK_ROWS, H), jnp.float32),
            pltpu.SemaphoreType.DMA,
            pltpu.SemaphoreType.DMA,
        ],
    )(c_tab, idx)

    return out_flat.reshape(B, L, H)


# speedup vs baseline: 6.6414x; 1.0067x over previous
import jax
import jax.numpy as jnp
from jax import lax
from jax.experimental import pallas as pl
from jax.experimental.pallas import tpu as pltpu
from jax.experimental.pallas import tpu_sc as plsc

H = 64
NE = 32
NA = 16
NV = NE * NA

NC = 2
NS = 16
NW = NC * NS

B_ROWS = 16384 * 50
ROWS_PER_W = B_ROWS // NW
IDX_MINOR = 128
IDX_ROWS_PER_W = ROWS_PER_W // IDX_MINOR
CHUNK_IDX_ROWS = 4
CHUNK_ROWS = CHUNK_IDX_ROWS * IDX_MINOR
N_CHUNKS = ROWS_PER_W // CHUNK_ROWS


def _prep_body(et_ref, at_ref, w_ref, b_ref, ent_ref, attr_ref, c_ref, idx_ref):
    e2 = jnp.dot(et_ref[...], w_ref[0:H, :], precision=lax.Precision.HIGHEST,
                 preferred_element_type=jnp.float32)
    a2 = jnp.dot(at_ref[...], w_ref[H:2 * H, :], precision=lax.Precision.HIGHEST,
                 preferred_element_type=jnp.float32)
    r = lax.broadcasted_iota(jnp.int32, (NV, NE), 0) // NA
    c = lax.broadcasted_iota(jnp.int32, (NV, NE), 1)
    oh_e = (r == c).astype(jnp.float32)
    r2 = lax.broadcasted_iota(jnp.int32, (NV, NA), 0) % NA
    c2 = lax.broadcasted_iota(jnp.int32, (NV, NA), 1)
    oh_a = (r2 == c2).astype(jnp.float32)
    c_ref[...] = (jnp.dot(oh_e, e2, precision=lax.Precision.HIGHEST,
                          preferred_element_type=jnp.float32)
                  + jnp.dot(oh_a, a2, precision=lax.Precision.HIGHEST,
                            preferred_element_type=jnp.float32)
                  + b_ref[...])
    idx_ref[...] = ent_ref[...] * NA + attr_ref[...]


def _sc_body(c_hbm, idx_hbm, out_hbm, idx_v, rows_v, gsems, ssems):
    wid = lax.axis_index("s") * NC + lax.axis_index("c")
    idx_row0 = wid * IDX_ROWS_PER_W
    out_row0 = wid * ROWS_PER_W

    pltpu.sync_copy(idx_hbm.at[pl.ds(idx_row0, IDX_ROWS_PER_W)], idx_v)

    def fire_gathers(c, slot):
        for j in range(CHUNK_IDX_ROWS):
            pltpu.async_copy(
                c_hbm.at[idx_v.at[c * CHUNK_IDX_ROWS + j]],
                rows_v.at[slot, pl.ds(j * IDX_MINOR, IDX_MINOR)],
                gsems[slot])

    def drain_gathers(slot):
        for j in range(CHUNK_IDX_ROWS):
            pltpu.make_async_copy(
                c_hbm.at[idx_v.at[j]],
                rows_v.at[slot, pl.ds(j * IDX_MINOR, IDX_MINOR)],
                gsems[slot]).wait()

    def fire_scatter(c, slot):
        pltpu.async_copy(
            rows_v.at[slot],
            out_hbm.at[pl.ds(out_row0 + c * CHUNK_ROWS, CHUNK_ROWS)],
            ssems[slot])

    def drain_scatter(slot):
        pltpu.make_async_copy(
            rows_v.at[slot],
            out_hbm.at[pl.ds(out_row0, CHUNK_ROWS)],
            ssems[slot]).wait()

    NBODY = N_CHUNKS // 3 - 1
    fire_gathers(0, 0)
    fire_gathers(1, 1)
    fire_gathers(2, 2)

    @pl.loop(0, NBODY)
    def _(i):
        c0 = i * 3
        for s in range(3):
            drain_gathers(s)
            fire_scatter(c0 + s, s)
        for s in range(3):
            drain_scatter(s)
            fire_gathers(c0 + 3 + s, s)

    c0 = NBODY * 3
    for s in range(3):
        drain_gathers(s)
        fire_scatter(c0 + s, s)
    for s in range(2):
        drain_scatter(s)
        fire_gathers(c0 + 3 + s, s)
    drain_scatter(2)
    for s in range(2):
        drain_gathers(s)
        fire_scatter(c0 + 3 + s, s)
    drain_scatter(0)
    drain_scatter(1)


def kernel(ent, attr, entity_table, attr_table, W, b):
    B, L = ent.shape
    ent32 = ent.astype(jnp.int32).reshape(B_ROWS // IDX_MINOR, IDX_MINOR)
    attr32 = attr.astype(jnp.int32).reshape(B_ROWS // IDX_MINOR, IDX_MINOR)
    b2 = b.reshape(1, H)

    c_tab, idx = pl.pallas_call(
        _prep_body,
        out_shape=(
            jax.ShapeDtypeStruct((NV, H), jnp.float32),
            jax.ShapeDtypeStruct((B_ROWS // IDX_MINOR, IDX_MINOR), jnp.int32),
        ),
    )(entity_table, attr_table, W, b2, ent32, attr32)

    mesh = plsc.VectorSubcoreMesh(core_axis_name="c", subcore_axis_name="s",
                                  num_cores=NC, num_subcores=NS)
    out_flat = pl.kernel(
        _sc_body,
        out_type=jax.ShapeDtypeStruct((B_ROWS, H), jnp.float32),
        mesh=mesh,
        compiler_params=pltpu.CompilerParams(use_tc_tiling_on_sc=False),
        scratch_types=[
            pltpu.VMEM((IDX_ROWS_PER_W, IDX_MINOR), jnp.int32),
            pltpu.VMEM((3, CHUNK_ROWS, H), jnp.float32),
            [pltpu.SemaphoreType.DMA] * 3,
            [pltpu.SemaphoreType.DMA] * 3,
        ],
    )(c_tab, idx)

    return out_flat.reshape(B, L, H)


# speedup vs baseline: 9.0303x; 1.3597x over previous
import jax
import jax.numpy as jnp
from jax import lax
from jax.experimental import pallas as pl
from jax.experimental.pallas import tpu as pltpu
from jax.experimental.pallas import tpu_sc as plsc

H = 64
NE = 32
NA = 16
NV = NE * NA

NC = 2
NS = 16
NW = NC * NS

B_ROWS = 16384 * 50
ROWS_PER_W = B_ROWS // NW
IDX_MINOR = 128
IDX_ROWS_PER_W = ROWS_PER_W // IDX_MINOR
CHUNK_IDX_ROWS = 4
CHUNK_ROWS = CHUNK_IDX_ROWS * IDX_MINOR
N_CHUNKS = ROWS_PER_W // CHUNK_ROWS
TAB_REP = 32


def _prep_body(et_ref, at_ref, w_ref, b_ref, ent_ref, attr_ref, c_ref, idx_ref):
    e2 = jnp.dot(et_ref[...], w_ref[0:H, :], precision=lax.Precision.HIGHEST,
                 preferred_element_type=jnp.float32)
    a2 = jnp.dot(at_ref[...], w_ref[H:2 * H, :], precision=lax.Precision.HIGHEST,
                 preferred_element_type=jnp.float32)
    r = lax.broadcasted_iota(jnp.int32, (NV, NE), 0) // NA
    c = lax.broadcasted_iota(jnp.int32, (NV, NE), 1)
    oh_e = (r == c).astype(jnp.float32)
    r2 = lax.broadcasted_iota(jnp.int32, (NV, NA), 0) % NA
    c2 = lax.broadcasted_iota(jnp.int32, (NV, NA), 1)
    oh_a = (r2 == c2).astype(jnp.float32)
    c_tab = (jnp.dot(oh_e, e2, precision=lax.Precision.HIGHEST,
                     preferred_element_type=jnp.float32)
             + jnp.dot(oh_a, a2, precision=lax.Precision.HIGHEST,
                       preferred_element_type=jnp.float32)
             + b_ref[...])
    for k in range(TAB_REP):
        c_ref[k * NV:(k + 1) * NV, :] = c_tab
    row = lax.broadcasted_iota(jnp.int32, idx_ref.shape, 0)
    rep = (row // IDX_ROWS_PER_W) % TAB_REP
    idx_ref[...] = ent_ref[...] * NA + attr_ref[...] + rep * NV


def _sc_body(c_hbm, idx_hbm, out_hbm, idx_v, rows_v, gsems, ssems):
    wid = lax.axis_index("s") * NC + lax.axis_index("c")
    idx_row0 = wid * IDX_ROWS_PER_W
    out_row0 = wid * ROWS_PER_W

    pltpu.sync_copy(idx_hbm.at[pl.ds(idx_row0, IDX_ROWS_PER_W)], idx_v)

    def fire_gathers(c, slot):
        for j in range(CHUNK_IDX_ROWS):
            pltpu.async_copy(
                c_hbm.at[idx_v.at[c * CHUNK_IDX_ROWS + j]],
                rows_v.at[slot, pl.ds(j * IDX_MINOR, IDX_MINOR)],
                gsems[slot])

    def drain_gathers(slot):
        for j in range(CHUNK_IDX_ROWS):
            pltpu.make_async_copy(
                c_hbm.at[idx_v.at[j]],
                rows_v.at[slot, pl.ds(j * IDX_MINOR, IDX_MINOR)],
                gsems[slot]).wait()

    def fire_scatter(c, slot):
        pltpu.async_copy(
            rows_v.at[slot],
            out_hbm.at[pl.ds(out_row0 + c * CHUNK_ROWS, CHUNK_ROWS)],
            ssems[slot])

    def drain_scatter(slot):
        pltpu.make_async_copy(
            rows_v.at[slot],
            out_hbm.at[pl.ds(out_row0, CHUNK_ROWS)],
            ssems[slot]).wait()

    NBODY = N_CHUNKS // 3 - 1
    fire_gathers(0, 0)
    fire_gathers(1, 1)
    fire_gathers(2, 2)

    @pl.loop(0, NBODY)
    def _(i):
        c0 = i * 3
        for s in range(3):
            drain_gathers(s)
            fire_scatter(c0 + s, s)
        for s in range(3):
            drain_scatter(s)
            fire_gathers(c0 + 3 + s, s)

    c0 = NBODY * 3
    for s in range(3):
        drain_gathers(s)
        fire_scatter(c0 + s, s)
    for s in range(2):
        drain_scatter(s)
        fire_gathers(c0 + 3 + s, s)
    drain_scatter(2)
    for s in range(2):
        drain_gathers(s)
        fire_scatter(c0 + 3 + s, s)
    drain_scatter(0)
    drain_scatter(1)


def kernel(ent, attr, entity_table, attr_table, W, b):
    B, L = ent.shape
    ent32 = ent.astype(jnp.int32).reshape(B_ROWS // IDX_MINOR, IDX_MINOR)
    attr32 = attr.astype(jnp.int32).reshape(B_ROWS // IDX_MINOR, IDX_MINOR)
    b2 = b.reshape(1, H)

    c_tab, idx = pl.pallas_call(
        _prep_body,
        out_shape=(
            jax.ShapeDtypeStruct((TAB_REP * NV, H), jnp.float32),
            jax.ShapeDtypeStruct((B_ROWS // IDX_MINOR, IDX_MINOR), jnp.int32),
        ),
    )(entity_table, attr_table, W, b2, ent32, attr32)

    mesh = plsc.VectorSubcoreMesh(core_axis_name="c", subcore_axis_name="s",
                                  num_cores=NC, num_subcores=NS)
    out_flat = pl.kernel(
        _sc_body,
        out_type=jax.ShapeDtypeStruct((B_ROWS, H), jnp.float32),
        mesh=mesh,
        compiler_params=pltpu.CompilerParams(use_tc_tiling_on_sc=False),
        scratch_types=[
            pltpu.VMEM((IDX_ROWS_PER_W, IDX_MINOR), jnp.int32),
            pltpu.VMEM((3, CHUNK_ROWS, H), jnp.float32),
            [pltpu.SemaphoreType.DMA] * 3,
            [pltpu.SemaphoreType.DMA] * 3,
        ],
    )(c_tab, idx)

    return out_flat.reshape(B, L, H)


# speedup vs baseline: 9.0621x; 1.0035x over previous
import jax
import jax.numpy as jnp
from jax import lax
from jax.experimental import pallas as pl
from jax.experimental.pallas import tpu as pltpu
from jax.experimental.pallas import tpu_sc as plsc

H = 64
NE = 32
NA = 16
NV = NE * NA
TAB_REP = 32

NC = 2
NS = 16
NW = NC * NS

B = 16384
L = 50
B_PER_W = B // NW
CB = 4
NBUF = 4
N_CHUNKS = B_PER_W // CB
NBODY = N_CHUNKS // NBUF - 1


def _prep_body(et_ref, at_ref, w_ref, b_ref, ent_ref, attr_ref, c_ref, idx_ref):
    e2 = jnp.dot(et_ref[...], w_ref[0:H, :], precision=lax.Precision.HIGHEST,
                 preferred_element_type=jnp.float32)
    a2 = jnp.dot(at_ref[...], w_ref[H:2 * H, :], precision=lax.Precision.HIGHEST,
                 preferred_element_type=jnp.float32)
    r = lax.broadcasted_iota(jnp.int32, (NV, NE), 0) // NA
    c = lax.broadcasted_iota(jnp.int32, (NV, NE), 1)
    oh_e = (r == c).astype(jnp.float32)
    r2 = lax.broadcasted_iota(jnp.int32, (NV, NA), 0) % NA
    c2 = lax.broadcasted_iota(jnp.int32, (NV, NA), 1)
    oh_a = (r2 == c2).astype(jnp.float32)
    c_tab = (jnp.dot(oh_e, e2, precision=lax.Precision.HIGHEST,
                     preferred_element_type=jnp.float32)
             + jnp.dot(oh_a, a2, precision=lax.Precision.HIGHEST,
                       preferred_element_type=jnp.float32)
             + b_ref[...])
    for k in range(TAB_REP):
        c_ref[k * NV:(k + 1) * NV, :] = c_tab
    row = lax.broadcasted_iota(jnp.int32, idx_ref.shape, 0)
    rep = (row // B_PER_W) % TAB_REP
    idx_ref[...] = ent_ref[...] * NA + attr_ref[...] + rep * NV


def _sc_body(c_hbm, idx_hbm, out_hbm, idx_v, rows_v, gsems, ssems):
    wid = lax.axis_index("s") * NC + lax.axis_index("c")
    b0_w = wid * B_PER_W

    pltpu.sync_copy(idx_hbm.at[pl.ds(b0_w, B_PER_W)], idx_v)

    def fire_gathers(c, slot):
        for j in range(CB):
            pltpu.async_copy(
                c_hbm.at[idx_v.at[c * CB + j]],
                rows_v.at[slot, j],
                gsems[slot])

    def drain_gathers(slot):
        for j in range(CB):
            pltpu.make_async_copy(
                c_hbm.at[idx_v.at[j]],
                rows_v.at[slot, j],
                gsems[slot]).wait()

    def fire_scatter(c, slot):
        pltpu.async_copy(
            rows_v.at[slot],
            out_hbm.at[pl.ds(b0_w + c * CB, CB)],
            ssems[slot])

    def drain_scatter(slot):
        pltpu.make_async_copy(
            rows_v.at[slot],
            out_hbm.at[pl.ds(b0_w, CB)],
            ssems[slot]).wait()

    for s in range(NBUF):
        fire_gathers(s, s)

    @pl.loop(0, NBODY)
    def _(i):
        c0 = i * NBUF
        for s in range(NBUF):
            drain_gathers(s)
            fire_scatter(c0 + s, s)
        for s in range(NBUF):
            drain_scatter(s)
            fire_gathers(c0 + NBUF + s, s)

    c0 = NBODY * NBUF
    for s in range(NBUF):
        drain_gathers(s)
        fire_scatter(c0 + s, s)
    for s in range(NBUF):
        drain_scatter(s)


def kernel(ent, attr, entity_table, attr_table, W, b):
    ent32 = ent.astype(jnp.int32)
    attr32 = attr.astype(jnp.int32)
    b2 = b.reshape(1, H)

    c_tab, idx = pl.pallas_call(
        _prep_body,
        out_shape=(
            jax.ShapeDtypeStruct((TAB_REP * NV, H), jnp.float32),
            jax.ShapeDtypeStruct((B, L), jnp.int32),
        ),
    )(entity_table, attr_table, W, b2, ent32, attr32)

    mesh = plsc.VectorSubcoreMesh(core_axis_name="c", subcore_axis_name="s",
                                  num_cores=NC, num_subcores=NS)
    out = pl.kernel(
        _sc_body,
        out_type=jax.ShapeDtypeStruct((B, L, H), jnp.float32),
        mesh=mesh,
        compiler_params=pltpu.CompilerParams(use_tc_tiling_on_sc=False),
        scratch_types=[
            pltpu.VMEM((B_PER_W, L), jnp.int32),
            pltpu.VMEM((NBUF, CB, L, H), jnp.float32),
            [pltpu.SemaphoreType.DMA] * NBUF,
            [pltpu.SemaphoreType.DMA] * NBUF,
        ],
    )(c_tab, idx)

    return out


# speedup vs baseline: 10.9148x; 1.2044x over previous
import jax
import jax.numpy as jnp
from jax import lax
from jax.experimental import pallas as pl
from jax.experimental.pallas import tpu as pltpu
from jax.experimental.pallas import tpu_sc as plsc

H = 64
NE = 32
NA = 16
NV = NE * NA
TAB_REP = 32

NC = 2
NS = 16
NW = NC * NS

B = 16384
L = 50
B_ROWS = B * L
ROWS_PER_W = B_ROWS // NW
IDX_MINOR = 128
IDX_ROWS_PER_W = ROWS_PER_W // IDX_MINOR
CHUNK_IDX_ROWS = 4
CHUNK_ROWS = CHUNK_IDX_ROWS * IDX_MINOR
N_CHUNKS = ROWS_PER_W // CHUNK_ROWS

TRB = 128


def _prep_body(et_ref, at_ref, w_ref, b_ref, ent_ref, attr_ref, c_ref, idx_ref):
    e2 = jnp.dot(et_ref[...], w_ref[0:H, :], precision=lax.Precision.HIGHEST,
                 preferred_element_type=jnp.float32)
    a2 = jnp.dot(at_ref[...], w_ref[H:2 * H, :], precision=lax.Precision.HIGHEST,
                 preferred_element_type=jnp.float32)
    r = lax.broadcasted_iota(jnp.int32, (NV, NE), 0) // NA
    c = lax.broadcasted_iota(jnp.int32, (NV, NE), 1)
    oh_e = (r == c).astype(jnp.float32)
    r2 = lax.broadcasted_iota(jnp.int32, (NV, NA), 0) % NA
    c2 = lax.broadcasted_iota(jnp.int32, (NV, NA), 1)
    oh_a = (r2 == c2).astype(jnp.float32)
    c_tab = (jnp.dot(oh_e, e2, precision=lax.Precision.HIGHEST,
                     preferred_element_type=jnp.float32)
             + jnp.dot(oh_a, a2, precision=lax.Precision.HIGHEST,
                       preferred_element_type=jnp.float32)
             + b_ref[...])
    for k in range(TAB_REP):
        c_ref[k * NV:(k + 1) * NV, :] = c_tab
    row = lax.broadcasted_iota(jnp.int32, idx_ref.shape, 0)
    rep = (row // IDX_ROWS_PER_W) % TAB_REP
    idx_ref[...] = ent_ref[...] * NA + attr_ref[...] + rep * NV


def _sc_body(c_hbm, idx_hbm, out_hbm, idx_v, rows_v, gsems, ssems):
    wid = lax.axis_index("s") * NC + lax.axis_index("c")
    idx_row0 = wid * IDX_ROWS_PER_W
    out_row0 = wid * ROWS_PER_W

    pltpu.sync_copy(idx_hbm.at[pl.ds(idx_row0, IDX_ROWS_PER_W)], idx_v)

    def fire_gathers(c, slot):
        for j in range(CHUNK_IDX_ROWS):
            pltpu.async_copy(
                c_hbm.at[idx_v.at[c * CHUNK_IDX_ROWS + j]],
                rows_v.at[slot, pl.ds(j * IDX_MINOR, IDX_MINOR)],
                gsems[slot])

    def drain_gathers(slot):
        for j in range(CHUNK_IDX_ROWS):
            pltpu.make_async_copy(
                c_hbm.at[idx_v.at[j]],
                rows_v.at[slot, pl.ds(j * IDX_MINOR, IDX_MINOR)],
                gsems[slot]).wait()

    def fire_scatter(c, slot):
        pltpu.async_copy(
            rows_v.at[slot],
            out_hbm.at[pl.ds(out_row0 + c * CHUNK_ROWS, CHUNK_ROWS)],
            ssems[slot])

    def drain_scatter(slot):
        pltpu.make_async_copy(
            rows_v.at[slot],
            out_hbm.at[pl.ds(out_row0, CHUNK_ROWS)],
            ssems[slot]).wait()

    NBODY = N_CHUNKS // 3 - 1
    fire_gathers(0, 0)
    fire_gathers(1, 1)
    fire_gathers(2, 2)

    @pl.loop(0, NBODY)
    def _(i):
        c0 = i * 3
        for s in range(3):
            drain_gathers(s)
            fire_scatter(c0 + s, s)
        for s in range(3):
            drain_scatter(s)
            fire_gathers(c0 + 3 + s, s)

    c0 = NBODY * 3
    for s in range(3):
        drain_gathers(s)
        fire_scatter(c0 + s, s)
    for s in range(2):
        drain_scatter(s)
        fire_gathers(c0 + 3 + s, s)
    drain_scatter(2)
    for s in range(2):
        drain_gathers(s)
        fire_scatter(c0 + 3 + s, s)
    drain_scatter(0)
    drain_scatter(1)


def _tr_body(x_ref, o_ref):
    o_ref[...] = x_ref[...].T


def kernel(ent, attr, entity_table, attr_table, W, b):
    ent32 = ent.astype(jnp.int32).reshape(B_ROWS // IDX_MINOR, IDX_MINOR)
    attr32 = attr.astype(jnp.int32).reshape(B_ROWS // IDX_MINOR, IDX_MINOR)
    b2 = b.reshape(1, H)

    c_tab, idx = pl.pallas_call(
        _prep_body,
        out_shape=(
            jax.ShapeDtypeStruct((TAB_REP * NV, H), jnp.float32),
            jax.ShapeDtypeStruct((B_ROWS // IDX_MINOR, IDX_MINOR), jnp.int32),
        ),
    )(entity_table, attr_table, W, b2, ent32, attr32)

    mesh = plsc.VectorSubcoreMesh(core_axis_name="c", subcore_axis_name="s",
                                  num_cores=NC, num_subcores=NS)
    tok = pl.kernel(
        _sc_body,
        out_type=jax.ShapeDtypeStruct((B_ROWS, H), jnp.float32),
        mesh=mesh,
        compiler_params=pltpu.CompilerParams(use_tc_tiling_on_sc=False),
        scratch_types=[
            pltpu.VMEM((IDX_ROWS_PER_W, IDX_MINOR), jnp.int32),
            pltpu.VMEM((3, CHUNK_ROWS, H), jnp.float32),
            [pltpu.SemaphoreType.DMA] * 3,
            [pltpu.SemaphoreType.DMA] * 3,
        ],
    )(c_tab, idx)

    x = tok.reshape(B, L * H)
    out2d = pl.pallas_call(
        _tr_body,
        grid=(B // TRB,),
        in_specs=[pl.BlockSpec((TRB, L * H), lambda i: (i, 0))],
        out_specs=pl.BlockSpec((L * H, TRB), lambda i: (0, i)),
        out_shape=jax.ShapeDtypeStruct((L * H, B), jnp.float32),
    )(x)
    return out2d.reshape(L, H, B).transpose(2, 0, 1)
